# Initial kernel scaffold; baseline (speedup 1.0000x reference)
import jax
import jax.numpy as jnp
from jax.experimental import pallas as pl


def kernel(compound_diseases, compound_phenotypes, compound_subcellular_locations, protein_diseases, protein_phenotypes, protein_subcellular_locations, disease_table, phenotype_table, sub_table, W1, b1, W2, b2, W3, b3):
    raise NotImplementedError("write your pallas kernel here")


# speedup vs baseline: 25.8847x; 25.8847x over previous
import functools

import jax
import jax.numpy as jnp
from jax import lax
from jax.experimental import pallas as pl
from jax.experimental.pallas import tpu as pltpu
from jax.experimental.pallas import tpu_sc as plsc

_B = 16384
_L = 200
_LS = 20
_LSP = 24
_DD, _DP, _DS = 32, 16, 16
_H1, _H2 = 128, 64
_F = 2 * (_DD + _DP + _DS)

_NC, _NS = 2, 16
_NW = _NC * _NS
_BPW = _B // _NW
_CE = 32
_NCHUNK = _BPW // _CE
_S0, _S1 = 128, 72


def _features_sc(cd, cp, cs, pd, pp, ps, dis_t, phe_t, sub_t):
    mesh = plsc.VectorSubcoreMesh(core_axis_name="c", subcore_axis_name="s")

    @functools.partial(
        pl.kernel,
        mesh=mesh,
        compiler_params=pltpu.CompilerParams(use_tc_tiling_on_sc=False),
        out_type=jax.ShapeDtypeStruct((_B, _F), jnp.float32),
        scratch_types=[
            pltpu.VMEM((_CE, _L), jnp.int32),
            pltpu.VMEM((_CE, _L), jnp.int32),
            pltpu.VMEM((_CE, _LSP), jnp.int32),
            pltpu.VMEM((_CE, _L), jnp.int32),
            pltpu.VMEM((_CE, _L), jnp.int32),
            pltpu.VMEM((_CE, _LSP), jnp.int32),
            pltpu.VMEM((_L, _DD), jnp.float32),
            pltpu.VMEM((_L, _DP), jnp.float32),
            pltpu.VMEM((_LSP, _DS), jnp.float32),
            pltpu.VMEM((_L, _DD), jnp.float32),
            pltpu.VMEM((_L, _DP), jnp.float32),
            pltpu.VMEM((_LSP, _DS), jnp.float32),
            pltpu.VMEM((_CE, _F), jnp.float32),
            pltpu.SemaphoreType.DMA,
            pltpu.SemaphoreType.DMA,
        ],
    )
    def feat_kernel(cd_h, cp_h, cs_h, pd_h, pp_h, ps_h, dis_h, phe_h, sub_h,
                    out_h, cd_i, cp_i, cs_i, pd_i, pp_i, ps_i,
                    cdb, cpb, csb, pdb, ppb, psb, feat, semg, sems):
        wid = lax.axis_index("s") * _NC + lax.axis_index("c")
        base = wid * _BPW

        def issue(e):
            cps = []
            for idx_i, buf, tbl in ((cd_i, cdb, dis_h), (cp_i, cpb, phe_h),
                                    (pd_i, pdb, dis_h), (pp_i, ppb, phe_h)):
                cps.append(pltpu.async_copy(
                    tbl.at[idx_i.at[e, pl.ds(0, _S0)]],
                    buf.at[pl.ds(0, _S0)], semg))
                cps.append(pltpu.async_copy(
                    tbl.at[idx_i.at[e, pl.ds(_S0, _S1)]],
                    buf.at[pl.ds(_S0, _S1)], semg))
            cps.append(pltpu.async_copy(sub_h.at[cs_i.at[e]], csb, semg))
            cps.append(pltpu.async_copy(sub_h.at[ps_i.at[e]], psb, semg))
            return cps

        def reduce_big(buf, e, col, d):
            ngrp = d // 16
            nacc = 4

            def body(j, accs):
                out = []
                for i in range(nacc):
                    row = j * nacc + i
                    for g in range(ngrp):
                        out.append(accs[i * ngrp + g]
                                   + buf[row, pl.ds(g * 16, 16)])
                return tuple(out)

            z = jnp.zeros((16,), jnp.float32)
            accs = lax.fori_loop(0, _L // nacc, body,
                                 tuple(z for _ in range(nacc * ngrp)))
            for g in range(ngrp):
                tot = ((accs[g] + accs[ngrp + g])
                       + (accs[2 * ngrp + g] + accs[3 * ngrp + g]))
                feat[e, pl.ds(col + g * 16, 16)] = tot * (1.0 / _L)

        def reduce_sub(buf, e, col):
            accs = [jnp.zeros((16,), jnp.float32) for _ in range(4)]
            for j in range(_LS):
                accs[j % 4] = accs[j % 4] + buf[j, pl.ds(0, 16)]
            tot = (accs[0] + accs[1]) + (accs[2] + accs[3])
            feat[e, pl.ds(col, 16)] = tot * (1.0 / _LS)

        def reduce_all(e):
            reduce_big(cdb, e, 0, _DD)
            reduce_big(cpb, e, _DD, _DP)
            reduce_sub(csb, e, _DD + _DP)
            reduce_big(pdb, e, 64, _DD)
            reduce_big(ppb, e, 64 + _DD, _DP)
            reduce_sub(psb, e, 64 + _DD + _DP)

        def chunk_body(c, _):
            cbase = base + c * _CE
            scp = []
            for src, dst in ((cd_h, cd_i), (cp_h, cp_i), (cs_h, cs_i),
                             (pd_h, pd_i), (pp_h, pp_i), (ps_h, ps_i)):
                scp.append(pltpu.async_copy(
                    src.at[pl.ds(cbase, _CE)], dst, sems))
            for d in scp:
                d.wait()

            def elem_body(e, __):
                for d in issue(e):
                    d.wait()
                reduce_all(e)
                return 0

            lax.fori_loop(0, _CE, elem_body, 0)
            pltpu.sync_copy(feat, out_h.at[pl.ds(cbase, _CE)])
            return 0

        lax.fori_loop(0, _NCHUNK, chunk_body, 0)

    return feat_kernel(cd, cp, cs, pd, pp, ps, dis_t, phe_t, sub_t)


def _mlp_tc(feat, w1, b1, w2, b2, w3t, b3):
    blk = 1024

    def body(x_ref, w1_ref, b1_ref, w2_ref, b2_ref, w3t_ref, b3_ref, o_ref):
        x = x_ref[...]
        h = jnp.dot(x, w1_ref[...], preferred_element_type=jnp.float32)
        h = h + b1_ref[...]
        h = jnp.where(h > 0, h, h * 0.01)
        h = jnp.dot(h, w2_ref[...], preferred_element_type=jnp.float32)
        h = h + b2_ref[...]
        h = jnp.where(h > 0, h, h * 0.01)
        o = jnp.sum(h * w3t_ref[...], axis=1, keepdims=True) + b3_ref[...]
        o_ref[...] = o

    return pl.pallas_call(
        body,
        grid=(_B // blk,),
        in_specs=[
            pl.BlockSpec((blk, _F), lambda i: (i, 0)),
            pl.BlockSpec((_F, _H1), lambda i: (0, 0)),
            pl.BlockSpec((1, _H1), lambda i: (0, 0)),
            pl.BlockSpec((_H1, _H2), lambda i: (0, 0)),
            pl.BlockSpec((1, _H2), lambda i: (0, 0)),
            pl.BlockSpec((1, _H2), lambda i: (0, 0)),
            pl.BlockSpec((1, 1), lambda i: (0, 0)),
        ],
        out_specs=pl.BlockSpec((blk, 1), lambda i: (i, 0)),
        out_shape=jax.ShapeDtypeStruct((_B, 1), jnp.float32),
    )(feat, w1, b1, w2, b2, w3t, b3)


def kernel(compound_diseases, compound_phenotypes,
           compound_subcellular_locations, protein_diseases,
           protein_phenotypes, protein_subcellular_locations,
           disease_table, phenotype_table, sub_table,
           W1, b1, W2, b2, W3, b3):
    cd = compound_diseases.astype(jnp.int32)
    cp = compound_phenotypes.astype(jnp.int32)
    pd = protein_diseases.astype(jnp.int32)
    pp = protein_phenotypes.astype(jnp.int32)
    cs = jnp.pad(compound_subcellular_locations.astype(jnp.int32),
                 ((0, 0), (0, _LSP - _LS)))
    ps = jnp.pad(protein_subcellular_locations.astype(jnp.int32),
                 ((0, 0), (0, _LSP - _LS)))

    feat = _features_sc(cd, cp, cs, pd, pp, ps,
                        disease_table, phenotype_table, sub_table)
    return _mlp_tc(feat, W1, b1.reshape(1, _H1), W2, b2.reshape(1, _H2),
                   W3.reshape(1, _H2), b3.reshape(1, 1))


# speedup vs baseline: 25.8984x; 1.0005x over previous
import functools

import jax
import jax.numpy as jnp
from jax import lax
from jax.experimental import pallas as pl
from jax.experimental.pallas import tpu as pltpu
from jax.experimental.pallas import tpu_sc as plsc

_B = 16384
_L = 200
_LS = 20
_LSP = 24
_DD, _DP, _DS = 32, 16, 16
_H1, _H2 = 128, 64
_F = 2 * (_DD + _DP + _DS)

_NC, _NS = 2, 16
_NW = _NC * _NS
_BPW = _B // _NW
_CE = 32
_NCHUNK = _BPW // _CE
_S0, _S1 = 128, 72


def _features_sc(cd, cp, cs, pd, pp, ps, dis_t, phe_t, sub_t):
    mesh = plsc.VectorSubcoreMesh(core_axis_name="c", subcore_axis_name="s")

    @functools.partial(
        pl.kernel,
        mesh=mesh,
        compiler_params=pltpu.CompilerParams(use_tc_tiling_on_sc=False),
        out_type=jax.ShapeDtypeStruct((_B, _F), jnp.float32),
        scratch_types=[
            pltpu.VMEM((_CE, _L), jnp.int32),
            pltpu.VMEM((_CE, _L), jnp.int32),
            pltpu.VMEM((_CE, _LSP), jnp.int32),
            pltpu.VMEM((_CE, _L), jnp.int32),
            pltpu.VMEM((_CE, _L), jnp.int32),
            pltpu.VMEM((_CE, _LSP), jnp.int32),
            pltpu.VMEM((2, _L, _DD), jnp.float32),
            pltpu.VMEM((2, _L, _DP), jnp.float32),
            pltpu.VMEM((2, _LSP, _DS), jnp.float32),
            pltpu.VMEM((2, _L, _DD), jnp.float32),
            pltpu.VMEM((2, _L, _DP), jnp.float32),
            pltpu.VMEM((2, _LSP, _DS), jnp.float32),
            pltpu.VMEM((_CE, _F), jnp.float32),
            pltpu.SemaphoreType.DMA,
            pltpu.SemaphoreType.DMA,
            pltpu.SemaphoreType.DMA,
        ],
    )
    def feat_kernel(cd_h, cp_h, cs_h, pd_h, pp_h, ps_h, dis_h, phe_h, sub_h,
                    out_h, cd_i, cp_i, cs_i, pd_i, pp_i, ps_i,
                    cdb, cpb, csb, pdb, ppb, psb, feat, sem0, sem1, sems):
        wid = lax.axis_index("s") * _NC + lax.axis_index("c")
        base = wid * _BPW

        def issue(e, si, sem):
            for idx_i, buf, tbl in ((cd_i, cdb, dis_h), (cp_i, cpb, phe_h),
                                    (pd_i, pdb, dis_h), (pp_i, ppb, phe_h)):
                pltpu.async_copy(tbl.at[idx_i.at[e, pl.ds(0, _S0)]],
                                 buf.at[si, pl.ds(0, _S0)], sem)
                pltpu.async_copy(tbl.at[idx_i.at[e, pl.ds(_S0, _S1)]],
                                 buf.at[si, pl.ds(_S0, _S1)], sem)
            pltpu.async_copy(sub_h.at[cs_i.at[e]], csb.at[si], sem)
            pltpu.async_copy(sub_h.at[ps_i.at[e]], psb.at[si], sem)

        def drain(si, sem):
            pltpu.make_async_copy(dis_h.at[pl.ds(0, _L)], cdb.at[si], sem).wait()
            pltpu.make_async_copy(phe_h.at[pl.ds(0, _L)], cpb.at[si], sem).wait()
            pltpu.make_async_copy(dis_h.at[pl.ds(0, _L)], pdb.at[si], sem).wait()
            pltpu.make_async_copy(phe_h.at[pl.ds(0, _L)], ppb.at[si], sem).wait()
            pltpu.make_async_copy(sub_h.at[pl.ds(0, _LSP)], csb.at[si], sem).wait()
            pltpu.make_async_copy(sub_h.at[pl.ds(0, _LSP)], psb.at[si], sem).wait()

        def reduce_big(buf, si, e, col, d):
            ngrp = d // 16
            nacc = 4

            def body(j, accs):
                out = []
                for i in range(nacc):
                    row = j * nacc + i
                    for g in range(ngrp):
                        out.append(accs[i * ngrp + g]
                                   + buf[si, row, pl.ds(g * 16, 16)])
                return tuple(out)

            z = jnp.zeros((16,), jnp.float32)
            accs = lax.fori_loop(0, _L // nacc, body,
                                 tuple(z for _ in range(nacc * ngrp)))
            for g in range(ngrp):
                tot = ((accs[g] + accs[ngrp + g])
                       + (accs[2 * ngrp + g] + accs[3 * ngrp + g]))
                feat[e, pl.ds(col + g * 16, 16)] = tot * (1.0 / _L)

        def reduce_sub(buf, si, e, col):
            accs = [jnp.zeros((16,), jnp.float32) for _ in range(4)]
            for j in range(_LS):
                accs[j % 4] = accs[j % 4] + buf[si, j, pl.ds(0, 16)]
            tot = (accs[0] + accs[1]) + (accs[2] + accs[3])
            feat[e, pl.ds(col, 16)] = tot * (1.0 / _LS)

        def reduce_all(si, e):
            reduce_big(cdb, si, e, 0, _DD)
            reduce_big(cpb, si, e, _DD, _DP)
            reduce_sub(csb, si, e, _DD + _DP)
            reduce_big(pdb, si, e, 64, _DD)
            reduce_big(ppb, si, e, 64 + _DD, _DP)
            reduce_sub(psb, si, e, 64 + _DD + _DP)

        def chunk_body(c, _):
            cbase = base + c * _CE
            scp = []
            for src, dst in ((cd_h, cd_i), (cp_h, cp_i), (cs_h, cs_i),
                             (pd_h, pd_i), (pp_h, pp_i), (ps_h, ps_i)):
                scp.append(pltpu.async_copy(
                    src.at[pl.ds(cbase, _CE)], dst, sems))
            for d in scp:
                d.wait()

            issue(0, 0, sem0)

            def pair_body(k2, __):
                e = 2 * k2
                issue(e + 1, 1, sem1)
                drain(0, sem0)
                reduce_all(0, e)

                @pl.when(k2 + 1 < _CE // 2)
                def _():
                    issue(e + 2, 0, sem0)

                drain(1, sem1)
                reduce_all(1, e + 1)
                return 0

            lax.fori_loop(0, _CE // 2, pair_body, 0)
            pltpu.sync_copy(feat, out_h.at[pl.ds(cbase, _CE)])
            return 0

        lax.fori_loop(0, _NCHUNK, chunk_body, 0)

    return feat_kernel(cd, cp, cs, pd, pp, ps, dis_t, phe_t, sub_t)


def _mlp_tc(feat, w1, b1, w2, b2, w3t, b3):
    blk = 1024

    def body(x_ref, w1_ref, b1_ref, w2_ref, b2_ref, w3t_ref, b3_ref, o_ref):
        x = x_ref[...]
        h = jnp.dot(x, w1_ref[...], preferred_element_type=jnp.float32)
        h = h + b1_ref[...]
        h = jnp.where(h > 0, h, h * 0.01)
        h = jnp.dot(h, w2_ref[...], preferred_element_type=jnp.float32)
        h = h + b2_ref[...]
        h = jnp.where(h > 0, h, h * 0.01)
        o = jnp.sum(h * w3t_ref[...], axis=1, keepdims=True) + b3_ref[...]
        o_ref[...] = o

    return pl.pallas_call(
        body,
        grid=(_B // blk,),
        in_specs=[
            pl.BlockSpec((blk, _F), lambda i: (i, 0)),
            pl.BlockSpec((_F, _H1), lambda i: (0, 0)),
            pl.BlockSpec((1, _H1), lambda i: (0, 0)),
            pl.BlockSpec((_H1, _H2), lambda i: (0, 0)),
            pl.BlockSpec((1, _H2), lambda i: (0, 0)),
            pl.BlockSpec((1, _H2), lambda i: (0, 0)),
            pl.BlockSpec((1, 1), lambda i: (0, 0)),
        ],
        out_specs=pl.BlockSpec((blk, 1), lambda i: (i, 0)),
        out_shape=jax.ShapeDtypeStruct((_B, 1), jnp.float32),
    )(feat, w1, b1, w2, b2, w3t, b3)


def kernel(compound_diseases, compound_phenotypes,
           compound_subcellular_locations, protein_diseases,
           protein_phenotypes, protein_subcellular_locations,
           disease_table, phenotype_table, sub_table,
           W1, b1, W2, b2, W3, b3):
    cd = compound_diseases.astype(jnp.int32)
    cp = compound_phenotypes.astype(jnp.int32)
    pd = protein_diseases.astype(jnp.int32)
    pp = protein_phenotypes.astype(jnp.int32)
    cs = jnp.pad(compound_subcellular_locations.astype(jnp.int32),
                 ((0, 0), (0, _LSP - _LS)))
    ps = jnp.pad(protein_subcellular_locations.astype(jnp.int32),
                 ((0, 0), (0, _LSP - _LS)))

    feat = _features_sc(cd, cp, cs, pd, pp, ps,
                        disease_table, phenotype_table, sub_table)
    return _mlp_tc(feat, W1, b1.reshape(1, _H1), W2, b2.reshape(1, _H2),
                   W3.reshape(1, _H2), b3.reshape(1, 1))
